# baseline (speedup 1.0000x reference)
import functools

import jax
import jax.numpy as jnp
from jax import lax
from jax.experimental import pallas as pl
from jax.experimental.pallas import tpu as pltpu
from jax.experimental.pallas import tpu_sc as plsc

VOCAB = 1000000
DIM = 64
B = 16384
NEG = 20

NC = 2
NS = 16
NW = NC * NS
BPW = B // NW
RCH = 16
NGRP = BPW // RCH

_mesh = plsc.VectorSubcoreMesh(core_axis_name="c", subcore_axis_name="s")


@functools.partial(
    pl.kernel,
    mesh=_mesh,
    out_type=[
        jax.ShapeDtypeStruct((B, DIM), jnp.float32),
        jax.ShapeDtypeStruct((B, DIM), jnp.float32),
        jax.ShapeDtypeStruct((B, NEG, DIM), jnp.float32),
    ],
    scratch_types=[
        pltpu.VMEM((BPW, NEG), jnp.int32),
        pltpu.VMEM((BPW, 1), jnp.int32),
        pltpu.VMEM((BPW, 1), jnp.int32),
        pltpu.VMEM((RCH, NEG, DIM), jnp.float32),
        pltpu.VMEM((RCH, NEG, DIM), jnp.float32),
        pltpu.VMEM((RCH, 1, DIM), jnp.float32),
        pltpu.VMEM((RCH, 1, DIM), jnp.float32),
        pltpu.VMEM((RCH, 1, DIM), jnp.float32),
        pltpu.VMEM((RCH, 1, DIM), jnp.float32),
        pltpu.SemaphoreType.DMA,
        pltpu.SemaphoreType.DMA,
        pltpu.SemaphoreType.DMA,
        pltpu.SemaphoreType.DMA,
        pltpu.SemaphoreType.DMA,
        pltpu.SemaphoreType.DMA,
        pltpu.SemaphoreType.DMA,
        pltpu.SemaphoreType.DMA,
    ],
    compiler_params=pltpu.CompilerParams(use_tc_tiling_on_sc=False),
)
def _skipgram(iw, ow, ng, in_tab, out_tab, o1, o2, o3,
              ngslab, iwslab, owslab, gb0, gb1, ib0, ib1, pb0, pb1,
              isem, gs0, gs1, is0, is1, os3a, os3b, osw):
    wid = lax.axis_index("s") * NC + lax.axis_index("c")
    base = pl.multiple_of(wid * BPW, BPW)

    h1 = pltpu.async_copy(ng.at[pl.ds(base, BPW), :], ngslab, isem)
    h2 = pltpu.async_copy(ow.at[pl.ds(base, BPW), :], owslab, isem)
    h3 = pltpu.async_copy(iw.at[pl.ds(base, BPW), :], iwslab, isem)
    h1.wait(); h2.wait(); h3.wait()

    gbufs = (gb0, gb1)
    ibufs = (ib0, ib1)
    pbufs = (pb0, pb1)
    gsems = (gs0, gs1)
    isems = (is0, is1)
    o3sems = (os3a, os3b)

    def fire_group(g, p):
        hs = []
        for r in range(RCH):
            row = g * RCH + r
            hs.append(pltpu.async_copy(
                out_tab.at[ngslab.at[row]], gbufs[p].at[r], gsems[p]))
            hs.append(pltpu.async_copy(
                in_tab.at[iwslab.at[row]], ibufs[p].at[r], isems[p]))
            hs.append(pltpu.async_copy(
                out_tab.at[owslab.at[row]], pbufs[p].at[r], isems[p]))
        return hs

    def drain_o3(g, p):
        rb = base + g * RCH
        pltpu.make_async_copy(gbufs[p],
                              o3.at[pl.ds(rb, RCH), :, :], o3sems[p]).wait()

    def body(g, _):
        for p in range(2):
            @pl.when(lax.rem(g, 2) == p)
            def _():
                @pl.when(g >= 2)
                def _():
                    drain_o3(g - 2, p)
                hs = fire_group(g, p)
                for h in hs:
                    h.wait()
                rb = base + g * RCH
                pltpu.async_copy(gbufs[p], o3.at[pl.ds(rb, RCH), :, :],
                                 o3sems[p])
                hw = [pltpu.async_copy(ibufs[p].at[:, 0, :],
                                       o1.at[pl.ds(rb, RCH)], osw),
                      pltpu.async_copy(pbufs[p].at[:, 0, :],
                                       o2.at[pl.ds(rb, RCH)], osw)]
                for h in hw:
                    h.wait()
        return 0

    lax.fori_loop(0, NGRP, body, 0)

    drain_o3(NGRP - 2, (NGRP - 2) % 2)
    drain_o3(NGRP - 1, (NGRP - 1) % 2)


def kernel(input_word, output_word, neg_samples, in_table, out_table):
    return tuple(_skipgram(input_word.astype(jnp.int32),
                           output_word.astype(jnp.int32),
                           neg_samples.astype(jnp.int32),
                           in_table, out_table))


# speedup vs baseline: 1.0240x; 1.0240x over previous
import functools

import jax
import jax.numpy as jnp
from jax import lax
from jax.experimental import pallas as pl
from jax.experimental.pallas import tpu as pltpu
from jax.experimental.pallas import tpu_sc as plsc

VOCAB = 1000000
DIM = 64
B = 16384
NEG = 20

NC = 2
NS = 16
NW = NC * NS
BPW = B // NW
NJOBS = 2 + NEG
NBUF = 3
DEPTH = 2

_mesh = plsc.VectorSubcoreMesh(core_axis_name="c", subcore_axis_name="s")


@functools.partial(
    pl.kernel,
    mesh=_mesh,
    out_type=[
        jax.ShapeDtypeStruct((B, DIM), jnp.float32),
        jax.ShapeDtypeStruct((B, DIM), jnp.float32),
        jax.ShapeDtypeStruct((B, NEG, DIM), jnp.float32),
    ],
    scratch_types=[
        pltpu.VMEM((2 + NEG, BPW), jnp.int32),
        pltpu.VMEM((BPW, DIM), jnp.float32),
        pltpu.VMEM((BPW, DIM), jnp.float32),
        pltpu.VMEM((BPW, DIM), jnp.float32),
        pltpu.SemaphoreType.DMA,
        pltpu.SemaphoreType.DMA,
        pltpu.SemaphoreType.DMA,
        pltpu.SemaphoreType.DMA,
        pltpu.SemaphoreType.DMA,
        pltpu.SemaphoreType.DMA,
        pltpu.SemaphoreType.DMA,
    ],
    compiler_params=pltpu.CompilerParams(use_tc_tiling_on_sc=False),
)
def _skipgram(iwt, owt, ngt, in_tab, out_tab, o1, o2, o3,
              idx_v, nb0, nb1, nb2,
              isem, g0, g1, g2, w0, w1, w2):
    wid = lax.axis_index("s") * NC + lax.axis_index("c")
    base = pl.multiple_of(wid * BPW, BPW)

    nbufs = (nb0, nb1, nb2)
    gsems = (g0, g1, g2)
    wsems = (w0, w1, w2)

    stg = [pltpu.async_copy(iwt.at[:, pl.ds(base, BPW)],
                            idx_v.at[pl.ds(0, 1), :], isem),
           pltpu.async_copy(owt.at[:, pl.ds(base, BPW)],
                            idx_v.at[pl.ds(1, 1), :], isem),
           pltpu.async_copy(ngt.at[:, pl.ds(base, BPW)],
                            idx_v.at[pl.ds(2, NEG), :], isem)]
    for h in stg:
        h.wait()

    jobs = [(idx_v.at[0], in_tab, o1.at[pl.ds(base, BPW)]),
            (idx_v.at[1], out_tab, o2.at[pl.ds(base, BPW)])]
    for j in range(NEG):
        jobs.append((idx_v.at[2 + j], out_tab, o3.at[pl.ds(base, BPW), j, :]))

    def fire_gather(j):
        offs, tab, _ = jobs[j]
        return pltpu.async_copy(
            tab.at[offs], nbufs[j % NBUF], gsems[j % NBUF])

    def fire_writeback(j):
        _, _, dst = jobs[j]
        return pltpu.async_copy(nbufs[j % NBUF], dst, wsems[j % NBUF])

    gh = [None] * NJOBS
    wh = [None] * NJOBS
    for j in range(DEPTH):
        gh[j] = fire_gather(j)
    for j in range(NJOBS):
        gh[j].wait()
        wh[j] = fire_writeback(j)
        if j + DEPTH < NJOBS:
            if j + DEPTH >= NBUF:
                wh[j + DEPTH - NBUF].wait()
            gh[j + DEPTH] = fire_gather(j + DEPTH)
    for j in range(NJOBS - DEPTH - 1, NJOBS):
        wh[j].wait()


def kernel(input_word, output_word, neg_samples, in_table, out_table):
    iwt = input_word.astype(jnp.int32).reshape(1, B)
    owt = output_word.astype(jnp.int32).reshape(1, B)
    ngt = neg_samples.astype(jnp.int32).T
    return tuple(_skipgram(iwt, owt, ngt, in_table, out_table))
